# Initial kernel scaffold; baseline (speedup 1.0000x reference)
import jax
import jax.numpy as jnp
from jax.experimental import pallas as pl


def kernel(word_ids, age_ids, bmi_ids, cycle_len_ids, word_table, demo_table, gamma, beta):
    raise NotImplementedError("write your pallas kernel here")


# speedup vs baseline: 1.0575x; 1.0575x over previous
import functools

import jax
import jax.numpy as jnp
from jax import lax
from jax.experimental import pallas as pl
from jax.experimental.pallas import tpu as pltpu
from jax.experimental.pallas import tpu_sc as plsc

_VOCAB = 1000000
_DEMO_VOCAB = 1000
_H = 64
_B, _L = 4096, 200
_N = _B * _L
_NW = 32
_PER_W = _N // _NW
_C = 128
_NCHUNK = _PER_W // _C
_NLANES = 16


def _sc_body(wid_hbm, age_hbm, bmi_hbm, cyc_hbm, wt_hbm, demo_hbm, gb_hbm,
             out_hbm, idxw, idxa, idxb, idxc, rows, demo, gb_v, sem):
    worker = lax.axis_index("s") * 2 + lax.axis_index("c")
    pltpu.sync_copy(demo_hbm, demo)
    pltpu.sync_copy(gb_hbm, gb_v)
    lanes = lax.iota(jnp.int32, _NLANES)
    woff = worker * _PER_W

    def chunk(i, carry):
        base = woff + i * _C
        pltpu.sync_copy(wid_hbm.at[pl.ds(base, _C)], idxw)
        pltpu.sync_copy(age_hbm.at[pl.ds(base, _C)], idxa)
        pltpu.sync_copy(bmi_hbm.at[pl.ds(base, _C)], idxb)
        pltpu.sync_copy(cyc_hbm.at[pl.ds(base, _C)], idxc)
        pltpu.async_copy(wt_hbm.at[idxw], rows, sem).wait()

        for g in range(_C // _NLANES):
            tok = lanes + (g * _NLANES)
            a0 = idxa[pl.ds(g * _NLANES, _NLANES)] * _H
            b0 = idxb[pl.ds(g * _NLANES, _NLANES)] * _H
            c0 = idxc[pl.ds(g * _NLANES, _NLANES)] * _H

            def pass_a(h, sc):
                s, s2 = sc
                col = jnp.full((_NLANES,), h, jnp.int32)
                x = (plsc.load_gather(rows, [tok, col])
                     + plsc.load_gather(demo, [a0 + h])
                     + plsc.load_gather(demo, [b0 + h])
                     + plsc.load_gather(demo, [c0 + h]))
                plsc.store_scatter(rows, [tok, col], x)
                return (s + x, s2 + x * x)

            zero = jnp.zeros((_NLANES,), jnp.float32)
            s, s2 = lax.fori_loop(0, _H, pass_a, (zero, zero))
            mean = s * (1.0 / _H)
            var = s2 * (1.0 / _H) - mean * mean
            v = var + 1e-12
            y = plsc.bitcast(
                jnp.int32(0x5F3759DF) - (plsc.bitcast(v, jnp.int32) >> 1),
                jnp.float32)
            for _ in range(3):
                y = y * (1.5 - 0.5 * v * y * y)
            rstd = y

            def pass_b(h, _):
                col = jnp.full((_NLANES,), h, jnp.int32)
                x = plsc.load_gather(rows, [tok, col])
                out = (x - mean) * rstd * gb_v[h] + gb_v[h + _H]
                plsc.store_scatter(rows, [tok, col], out)
                return 0

            lax.fori_loop(0, _H, pass_b, 0)

        pltpu.sync_copy(rows, out_hbm.at[pl.ds(base, _C)])
        return carry

    lax.fori_loop(0, _NCHUNK, chunk, 0)


@jax.jit
def kernel(word_ids, age_ids, bmi_ids, cycle_len_ids, word_table, demo_table,
           gamma, beta):
    wid = word_ids.reshape(_N).astype(jnp.int32)
    age = age_ids.reshape(_N).astype(jnp.int32)
    bmi = bmi_ids.reshape(_N).astype(jnp.int32)
    cyc = cycle_len_ids.reshape(_N).astype(jnp.int32)
    demo_flat = demo_table.reshape(_DEMO_VOCAB * _H)
    gb = jnp.repeat(
        jnp.concatenate([gamma, beta]).astype(jnp.float32)[:, None],
        _NLANES, axis=1)

    mesh = plsc.VectorSubcoreMesh(core_axis_name="c", subcore_axis_name="s")
    run = pl.kernel(
        _sc_body,
        out_type=jax.ShapeDtypeStruct((_N, _H), jnp.float32),
        mesh=mesh,
        scratch_types=[
            pltpu.VMEM((_C,), jnp.int32),
            pltpu.VMEM((_C,), jnp.int32),
            pltpu.VMEM((_C,), jnp.int32),
            pltpu.VMEM((_C,), jnp.int32),
            pltpu.VMEM((_C, _H), jnp.float32),
            pltpu.VMEM((_DEMO_VOCAB * _H,), jnp.float32),
            pltpu.VMEM((2 * _H, _NLANES), jnp.float32),
            pltpu.SemaphoreType.DMA,
        ],
        compiler_params=pltpu.CompilerParams(
            needs_layout_passes=False, use_tc_tiling_on_sc=False),
    )
    out = run(wid, age, bmi, cyc, word_table, demo_flat, gb)
    return out.reshape(_B, _L, _H)
